# Initial kernel scaffold; baseline (speedup 1.0000x reference)
import jax
import jax.numpy as jnp
from jax.experimental import pallas as pl


def kernel(input, W, b):
    raise NotImplementedError("write your pallas kernel here")


# speedup vs baseline: 24.9011x; 24.9011x over previous
import functools

import jax
import jax.numpy as jnp
from jax.experimental import pallas as pl
from jax.experimental.pallas import tpu as pltpu

_G0 = 128
_G1 = 128
_E = _G0 + _G1
_K = 4
_NEG = float("-inf")


def _top4(x, width):
    iota = jax.lax.broadcasted_iota(jnp.int32, x.shape, 1)
    vals, idxs = [], []
    for _ in range(_K):
        m = jnp.max(x, axis=1, keepdims=True)
        is_max = x == m
        idx = jnp.min(jnp.where(is_max, iota, width), axis=1, keepdims=True)
        vals.append(m)
        idxs.append(idx)
        x = jnp.where(iota == idx, _NEG, x)
    return vals, idxs


def _gating_kernel(x_ref, w_ref, b_ref, ids_ref, logits_ref, wts_ref):
    x = x_ref[...]
    scores = jax.lax.dot_general(
        x, w_ref[...], (((1,), (1,)), ((), ())),
        preferred_element_type=jnp.float32,
    ) + b_ref[...]
    s0 = scores[:, :_G0]
    s1 = scores[:, _G0:]

    v0, i0 = _top4(s0, _G0)

    cand = jnp.concatenate([v0[b] + s1 for b in range(_K)], axis=1)
    v2, lin = _top4(cand, _K * _G1)

    ids_cols, logit_cols, exp_cols = [], [], []
    for t in range(_K):
        b_t = lin[t] // _G1
        j_t = lin[t] % _G1
        id0 = jnp.zeros_like(b_t)
        for b in range(_K):
            id0 = jnp.where(b_t == b, i0[b], id0)
        ids_cols.append(id0 * _G1 + j_t)
        logit_cols.append(v2[t])
        exp_cols.append(jnp.exp(v2[t] - v2[0]))

    denom = exp_cols[0] + exp_cols[1] + exp_cols[2] + exp_cols[3]
    ids_ref[...] = jnp.concatenate(ids_cols, axis=1)
    logits_ref[...] = jnp.concatenate(logit_cols, axis=1)
    wts_ref[...] = jnp.concatenate([e / denom for e in exp_cols], axis=1)


@functools.partial(jax.jit, static_argnames=())
def kernel(input, W, b):
    n, d = input.shape
    bt = 512
    grid = (n // bt,)
    out_specs = [
        pl.BlockSpec((bt, _K), lambda i: (i, 0)),
        pl.BlockSpec((bt, _K), lambda i: (i, 0)),
        pl.BlockSpec((bt, _K), lambda i: (i, 0)),
    ]
    ids, logits, wts = pl.pallas_call(
        _gating_kernel,
        grid=grid,
        in_specs=[
            pl.BlockSpec((bt, d), lambda i: (i, 0)),
            pl.BlockSpec((_E, d), lambda i: (0, 0)),
            pl.BlockSpec((_E,), lambda i: (0,)),
        ],
        out_specs=out_specs,
        out_shape=[
            jax.ShapeDtypeStruct((n, _K), jnp.int32),
            jax.ShapeDtypeStruct((n, _K), jnp.float32),
            jax.ShapeDtypeStruct((n, _K), jnp.float32),
        ],
    )(input, W, b)
    return ids, logits, wts


# speedup vs baseline: 25.8598x; 1.0385x over previous
import functools

import jax
import jax.numpy as jnp
from jax.experimental import pallas as pl
from jax.experimental.pallas import tpu as pltpu

_G0 = 128
_G1 = 128
_E = _G0 + _G1
_K = 4
_NEG = float("-inf")


def _top4(x, width):
    iota = jax.lax.broadcasted_iota(jnp.int32, x.shape, 1)
    vals, idxs = [], []
    for _ in range(_K):
        m = jnp.max(x, axis=1, keepdims=True)
        is_max = x == m
        idx = jnp.min(jnp.where(is_max, iota, width), axis=1, keepdims=True)
        vals.append(m)
        idxs.append(idx)
        x = jnp.where(iota == idx, _NEG, x)
    return vals, idxs


def _gating_kernel(x_ref, w_ref, b_ref, ids_ref, logits_ref, wts_ref):
    x = x_ref[...]
    scores = jax.lax.dot_general(
        x, w_ref[...], (((1,), (1,)), ((), ())),
        preferred_element_type=jnp.float32,
    ) + b_ref[...]
    s0 = scores[:, :_G0]
    s1 = scores[:, _G0:]

    v0, i0 = _top4(s0, _G0)

    cand = jnp.concatenate([v0[b] + s1 for b in range(_K)], axis=1)
    v2, lin = _top4(cand, _K * _G1)

    ids_cols, logit_cols, exp_cols = [], [], []
    for t in range(_K):
        b_t = lin[t] // _G1
        j_t = lin[t] % _G1
        id0 = jnp.zeros_like(b_t)
        for b in range(_K):
            id0 = jnp.where(b_t == b, i0[b], id0)
        ids_cols.append(id0 * _G1 + j_t)
        logit_cols.append(v2[t])
        exp_cols.append(jnp.exp(v2[t] - v2[0]))

    denom = exp_cols[0] + exp_cols[1] + exp_cols[2] + exp_cols[3]
    ids_ref[...] = jnp.concatenate(ids_cols, axis=1)
    logits_ref[...] = jnp.concatenate(logit_cols, axis=1)
    wts_ref[...] = jnp.concatenate([e / denom for e in exp_cols], axis=1)


@functools.partial(jax.jit, static_argnames=())
def kernel(input, W, b):
    n, d = input.shape
    bt = 1024
    grid = (n // bt,)
    out_specs = [
        pl.BlockSpec((bt, _K), lambda i: (i, 0)),
        pl.BlockSpec((bt, _K), lambda i: (i, 0)),
        pl.BlockSpec((bt, _K), lambda i: (i, 0)),
    ]
    ids, logits, wts = pl.pallas_call(
        _gating_kernel,
        grid=grid,
        in_specs=[
            pl.BlockSpec((bt, d), lambda i: (i, 0)),
            pl.BlockSpec((_E, d), lambda i: (0, 0)),
            pl.BlockSpec((_E,), lambda i: (0,)),
        ],
        out_specs=out_specs,
        out_shape=[
            jax.ShapeDtypeStruct((n, _K), jnp.int32),
            jax.ShapeDtypeStruct((n, _K), jnp.float32),
            jax.ShapeDtypeStruct((n, _K), jnp.float32),
        ],
    )(input, W, b)
    return ids, logits, wts


# speedup vs baseline: 46.7634x; 1.8083x over previous
import jax
import jax.numpy as jnp
from jax.experimental import pallas as pl

_G0 = 128
_G1 = 128
_E = _G0 + _G1
_K = 4
_NEG = float("-inf")


def _top4_rows(x):
    g = x.shape[0]
    iota = jax.lax.broadcasted_iota(jnp.int32, x.shape, 0)
    vals, idxs = [], []
    for _ in range(_K):
        m = jnp.max(x, axis=0, keepdims=True)
        is_max = x == m
        idx = jnp.min(jnp.where(is_max, iota, g), axis=0, keepdims=True)
        vals.append(m)
        idxs.append(idx)
        x = jnp.where(iota == idx, _NEG, x)
    return vals, idxs


def _gating_kernel(x_ref, w_ref, b_ref, ids_ref, logits_ref, wts_ref):
    scores = jax.lax.dot_general(
        w_ref[...], x_ref[...], (((1,), (1,)), ((), ())),
        preferred_element_type=jnp.float32,
    ) + b_ref[...]
    v0, i0 = _top4_rows(scores[:_G0, :])
    v1, i1 = _top4_rows(scores[_G0:, :])

    cand = jnp.concatenate(
        [v0[b] + v1[j] for b in range(_K) for j in range(_K)], axis=0)
    lin = jnp.concatenate(
        [b * _G1 + i1[j] for b in range(_K) for j in range(_K)], axis=0)
    eid = jnp.concatenate(
        [i0[b] * _G1 + i1[j] for b in range(_K) for j in range(_K)], axis=0)

    big = _K * _G1
    ids_rows, logit_rows, exp_rows = [], [], []
    for t in range(_K):
        m = jnp.max(cand, axis=0, keepdims=True)
        l = jnp.min(jnp.where(cand == m, lin, big), axis=0, keepdims=True)
        hit = lin == l
        ids_rows.append(jnp.sum(jnp.where(hit, eid, 0), axis=0, keepdims=True))
        logit_rows.append(m)
        exp_rows.append(jnp.exp(m - logit_rows[0]))
        cand = jnp.where(hit, _NEG, cand)

    denom = exp_rows[0] + exp_rows[1] + exp_rows[2] + exp_rows[3]
    ids_ref[...] = jnp.concatenate(ids_rows, axis=0)
    logits_ref[...] = jnp.concatenate(logit_rows, axis=0)
    wts_ref[...] = jnp.concatenate([e / denom for e in exp_rows], axis=0)


def kernel(input, W, b):
    n, d = input.shape
    bt = 1024
    grid = (n // bt,)
    ids_t, logits_t, wts_t = pl.pallas_call(
        _gating_kernel,
        grid=grid,
        in_specs=[
            pl.BlockSpec((bt, d), lambda i: (i, 0)),
            pl.BlockSpec((_E, d), lambda i: (0, 0)),
            pl.BlockSpec((_E, 1), lambda i: (0, 0)),
        ],
        out_specs=[
            pl.BlockSpec((_K, bt), lambda i: (0, i)),
            pl.BlockSpec((_K, bt), lambda i: (0, i)),
            pl.BlockSpec((_K, bt), lambda i: (0, i)),
        ],
        out_shape=[
            jax.ShapeDtypeStruct((_K, n), jnp.int32),
            jax.ShapeDtypeStruct((_K, n), jnp.float32),
            jax.ShapeDtypeStruct((_K, n), jnp.float32),
        ],
    )(input, W, b.reshape(_E, 1))
    return ids_t.T, logits_t.T, wts_t.T


kernel = jax.jit(kernel)
